# baseline (speedup 1.0000x reference)
import functools

import jax
import jax.numpy as jnp
from jax import lax
from jax.experimental import pallas as pl
from jax.experimental.pallas import tpu as pltpu
from jax.experimental.pallas import tpu_sc as plsc

BATCH = 4096
HIST = 200
D = 64
V = 1000000
NC = 2
NS = 16
NW = NC * NS
VP = 128
GS = 129
BB = BATCH // NW

_mesh = plsc.VectorSubcoreMesh(core_axis_name="c", subcore_axis_name="s")
_params = pltpu.CompilerParams(
    use_tc_tiling_on_sc=True, needs_layout_passes=False
)


@functools.partial(
    pl.kernel,
    mesh=_mesh,
    out_type=jax.ShapeDtypeStruct((HIST, D, BATCH), jnp.float32),
    scratch_types=[
        pltpu.VMEM((HIST, BB), jnp.int32),
        pltpu.VMEM((2, BB, GS), jnp.float32),
        pltpu.VMEM((2, D, GS), jnp.float32),
        pltpu.SemaphoreType.DMA,
        pltpu.SemaphoreType.DMA,
        pltpu.SemaphoreType.DMA,
        pltpu.SemaphoreType.DMA,
    ],
    compiler_params=_params,
)
def _gather_t(it_hbm, tp_hbm, out_hbm, idx_v, gbuf, obuf,
              gsem0, gsem1, wsem0, wsem1):
    cid = lax.axis_index("c")
    sid = lax.axis_index("s")
    wid = sid * NC + cid
    b0 = pl.multiple_of(wid * BB, BB)
    pltpu.sync_copy(it_hbm.at[:, pl.ds(b0, BB)], idx_v)

    rows = [lax.iota(jnp.int32, 16) + 16 * g for g in range(BB // 16)]

    def fire_gather(t, b, sem):
        pltpu.async_copy(
            tp_hbm.at[idx_v.at[t]], gbuf.at[b, :, pl.ds(0, VP)], sem
        )

    def drain_gather(b, sem):
        pltpu.make_async_copy(
            tp_hbm.at[idx_v.at[0]], gbuf.at[b, :, pl.ds(0, VP)], sem
        ).wait()

    def wait_write(b, sem):
        pltpu.make_async_copy(
            obuf.at[b, :, pl.ds(0, BB)], out_hbm.at[0, :, pl.ds(b0, BB)], sem
        ).wait()

    def transpose_block(b):
        def jrow(jo, _):
            for ji in range(8):
                j = jo * 8 + ji
                cols = jnp.full((16,), j, jnp.int32)
                for g in range(D // 16):
                    v = gbuf[b, j, pl.ds(16 * g, 16)]
                    plsc.store_scatter(obuf.at[b], [rows[g], cols], v)
            return 0
        lax.fori_loop(0, BB // 8, jrow, 0)

    fire_gather(0, 0, gsem0)

    def pair(p, _):
        fire_gather(2 * p + 1, 1, gsem1)
        drain_gather(0, gsem0)

        @pl.when(p >= 1)
        def _():
            wait_write(0, wsem0)
        transpose_block(0)
        pltpu.async_copy(
            obuf.at[0, :, pl.ds(0, BB)],
            out_hbm.at[2 * p, :, pl.ds(b0, BB)], wsem0
        )

        @pl.when(p < HIST // 2 - 1)
        def _():
            fire_gather(2 * p + 2, 0, gsem0)
        drain_gather(1, gsem1)

        @pl.when(p >= 1)
        def _():
            wait_write(1, wsem1)
        transpose_block(1)
        pltpu.async_copy(
            obuf.at[1, :, pl.ds(0, BB)],
            out_hbm.at[2 * p + 1, :, pl.ds(b0, BB)], wsem1
        )
        return 0

    lax.fori_loop(0, HIST // 2, pair, 0)
    wait_write(0, wsem0)
    wait_write(1, wsem1)


def kernel(items, table):
    items_t = items.astype(jnp.int32).T
    tp = jnp.concatenate(
        [table, jnp.zeros((V, VP - D), jnp.float32)], axis=1
    )
    out_t = _gather_t(items_t, tp)
    return jnp.transpose(out_t, (2, 0, 1))


# speedup vs baseline: 1.2559x; 1.2559x over previous
import functools

import jax
import jax.numpy as jnp
from jax import lax
from jax.experimental import pallas as pl
from jax.experimental.pallas import tpu as pltpu
from jax.experimental.pallas import tpu_sc as plsc

BATCH = 4096
HIST = 200
D = 64
NC = 2
NS = 16
NW = NC * NS
RB = BATCH // NW
G = 2
NG = RB // G
CHUNKS = ((0, 128), (128, 72))

_mesh = plsc.VectorSubcoreMesh(core_axis_name="c", subcore_axis_name="s")


@functools.partial(
    pl.kernel,
    mesh=_mesh,
    out_type=jax.ShapeDtypeStruct((BATCH, HIST, D), jnp.float32),
    scratch_types=[
        pltpu.VMEM((RB, HIST), jnp.int32),
        pltpu.VMEM((2, G, HIST, D), jnp.float32),
        pltpu.SemaphoreType.DMA,
        pltpu.SemaphoreType.DMA,
        pltpu.SemaphoreType.DMA,
        pltpu.SemaphoreType.DMA,
    ],
    compiler_params=pltpu.CompilerParams(use_tc_tiling_on_sc=False),
)
def _emb_lookup(idx_hbm, table_hbm, out_hbm, idx_v, rows_v, gsem0, gsem1,
                wsem0, wsem1):
    cid = lax.axis_index("c")
    sid = lax.axis_index("s")
    wid = sid * NC + cid
    row0 = wid * RB
    pltpu.sync_copy(idx_hbm.at[pl.ds(row0, RB)], idx_v)

    def fire_gathers(g, b, sem):
        for i in range(G):
            for off, n in CHUNKS:
                pltpu.async_copy(
                    table_hbm.at[idx_v.at[g * G + i, pl.ds(off, n)]],
                    rows_v.at[b, i, pl.ds(off, n)],
                    sem,
                )

    def drain_gathers(b, sem):
        for i in range(G):
            for off, n in CHUNKS:
                pltpu.make_async_copy(
                    table_hbm.at[idx_v.at[0, pl.ds(0, n)]],
                    rows_v.at[b, i, pl.ds(off, n)],
                    sem,
                ).wait()

    def wait_write(b, sem):
        pltpu.make_async_copy(
            rows_v.at[b], out_hbm.at[pl.ds(0, G)], sem
        ).wait()

    fire_gathers(0, 0, gsem0)

    def pair(p, _):
        @pl.when(p >= 1)
        def _():
            wait_write(1, wsem1)
        fire_gathers(2 * p + 1, 1, gsem1)
        drain_gathers(0, gsem0)
        pltpu.async_copy(
            rows_v.at[0], out_hbm.at[pl.ds(row0 + 2 * p * G, G)], wsem0
        )

        @pl.when(p < NG // 2 - 1)
        def _():
            wait_write(0, wsem0)
            fire_gathers(2 * p + 2, 0, gsem0)
        drain_gathers(1, gsem1)
        pltpu.async_copy(
            rows_v.at[1], out_hbm.at[pl.ds(row0 + (2 * p + 1) * G, G)], wsem1
        )
        return 0

    lax.fori_loop(0, NG // 2, pair, 0)
    wait_write(0, wsem0)
    wait_write(1, wsem1)


def kernel(items, table):
    return _emb_lookup(items.astype(jnp.int32), table)
